# baseline (speedup 1.0000x reference)
import jax
import jax.numpy as jnp
from jax.experimental import pallas as pl
from jax.experimental.pallas import tpu as pltpu

_BINS = 100
_IGNORE = 0
_R = 32


def _stat_block(ys_sref, x_ref, ys_ref, acc_ref):
    i = pl.program_id(0)

    @pl.when(i == 0)
    def _():
        acc_ref[...] = jnp.zeros_like(acc_ref)

    x = x_ref[...]
    ys = ys_ref[0, 0, :]
    nk = 16
    vv = x.shape[1]
    step = ((vv // nk) // 128) * 128
    cuts = [0] + [step * (k + 1) for k in range(nk - 1)] + [vv]
    spans = list(zip(cuts[:-1], cuts[1:]))
    mparts = [jnp.max(x[:, c0:c1], axis=1) for c0, c1 in spans]
    m = mparts[0]
    for mp in mparts[1:]:
        m = jnp.maximum(m, mp)
    s = sum(jnp.sum(jnp.exp(x[:, c0:c1] - m[:, None]), axis=1)
            for c0, c1 in spans)

    v = x.shape[1]
    v_al = (v // 128) * 128
    segs = []
    for r in range(_R):
        y_r = jnp.minimum(ys_sref[i * _R + r], v_al - 1)
        start = pl.multiple_of((y_r // 128) * 128, 128)
        segs.append(x_ref[pl.ds(r, 1), pl.ds(start, 128)])
    seg = jnp.concatenate(segs, axis=0)
    lane = jax.lax.broadcasted_iota(jnp.int32, (_R, 128), 1)
    main_lane = (jnp.minimum(ys, v_al - 1) % 128)[:, None]
    tgt = jnp.sum(jnp.where(lane == main_lane, seg, 0.0), axis=1)
    if v_al < v:
        tail = x[:, v_al:]
        tl = jax.lax.broadcasted_iota(jnp.int32, tail.shape, 1)
        tgt_tail = jnp.sum(jnp.where(tl == (ys - v_al)[:, None], tail, 0.0),
                           axis=1)
        tgt = jnp.where(ys >= v_al, tgt_tail, tgt)

    pv = jnp.exp(tgt - m) / s
    valid = (ys != _IGNORE).astype(jnp.float32)

    lanes_i = jax.lax.broadcasted_iota(jnp.int32, (_R, 128), 1)
    lanes_f = lanes_i.astype(jnp.float32)
    upper = pv[:, None] > lanes_f / _BINS
    lower = pv[:, None] < lanes_f + (1.0 / _BINS)
    mask = (upper & lower & (lanes_i < _BINS)).astype(jnp.float32) * valid[:, None]
    hist = jnp.sum(mask, axis=0)

    lane1 = jax.lax.iota(jnp.int32, 128)
    extra = jnp.where(lane1 == _BINS, jnp.sum(pv * valid),
                      jnp.where(lane1 == _BINS + 1, jnp.sum(valid), 0.0))
    acc_ref[0, :] += hist + extra


def kernel(decoder_out_att, ys_out_pad_att):
    b, t, v = decoder_out_att.shape
    n = b * t
    x = decoder_out_att.reshape(n, v)
    ys_flat = ys_out_pad_att.reshape(n)
    ys3 = ys_out_pad_att.reshape(n // _R, 1, _R)
    grid_spec = pltpu.PrefetchScalarGridSpec(
        num_scalar_prefetch=1,
        grid=(n // _R,),
        in_specs=[pl.BlockSpec((_R, v), lambda i, *_: (i, 0)),
                  pl.BlockSpec((1, 1, _R), lambda i, *_: (i, 0, 0))],
        out_specs=pl.BlockSpec((1, 128), lambda i, *_: (0, 0)),
    )
    acc = pl.pallas_call(
        _stat_block,
        grid_spec=grid_spec,
        out_shape=jax.ShapeDtypeStruct((1, 128), jnp.float32),
        compiler_params=pltpu.CompilerParams(dimension_semantics=("arbitrary",)),
    )(ys_flat, x, ys3)[0]
    mean = acc[_BINS] / jnp.maximum(acc[_BINS + 1], 1.0)
    return jnp.concatenate([mean[None], acc[:_BINS]])


# speedup vs baseline: 1.0342x; 1.0342x over previous
import jax
import jax.numpy as jnp
from jax.experimental import pallas as pl
from jax.experimental.pallas import tpu as pltpu

_BINS = 100
_IGNORE = 0
_R = 32


def _stat_block(ys_sref, x_ref, ys_ref, acc_ref):
    i = pl.program_id(0)

    @pl.when(i == 0)
    def _():
        acc_ref[...] = jnp.zeros_like(acc_ref)

    x = x_ref[...]
    ys = ys_ref[0, 0, :]
    nk = 8
    vv = x.shape[1]
    step = ((vv // nk) // 128) * 128
    cuts = [0] + [step * (k + 1) for k in range(nk - 1)] + [vv]
    spans = list(zip(cuts[:-1], cuts[1:]))
    mparts = [jnp.max(x[:, c0:c1], axis=1) for c0, c1 in spans]
    m = mparts[0]
    for mp in mparts[1:]:
        m = jnp.maximum(m, mp)
    s = sum(jnp.sum(jnp.exp(x[:, c0:c1] - m[:, None]), axis=1)
            for c0, c1 in spans)

    v = x.shape[1]
    v_al = (v // 128) * 128
    segs = []
    for r in range(_R):
        y_r = jnp.minimum(ys_sref[i * _R + r], v_al - 1)
        start = pl.multiple_of((y_r // 128) * 128, 128)
        segs.append(x_ref[pl.ds(r, 1), pl.ds(start, 128)])
    seg = jnp.concatenate(segs, axis=0)
    lane = jax.lax.broadcasted_iota(jnp.int32, (_R, 128), 1)
    main_lane = (jnp.minimum(ys, v_al - 1) % 128)[:, None]
    tgt = jnp.sum(jnp.where(lane == main_lane, seg, 0.0), axis=1)
    if v_al < v:
        tail = x[:, v_al:]
        tl = jax.lax.broadcasted_iota(jnp.int32, tail.shape, 1)
        tgt_tail = jnp.sum(jnp.where(tl == (ys - v_al)[:, None], tail, 0.0),
                           axis=1)
        tgt = jnp.where(ys >= v_al, tgt_tail, tgt)

    pv = jnp.exp(tgt - m) / s
    valid = (ys != _IGNORE).astype(jnp.float32)

    lanes_i = jax.lax.broadcasted_iota(jnp.int32, (_R, 128), 1)
    lanes_f = lanes_i.astype(jnp.float32)
    upper = pv[:, None] > lanes_f / _BINS
    lower = pv[:, None] < lanes_f + (1.0 / _BINS)
    mask = (upper & lower & (lanes_i < _BINS)).astype(jnp.float32) * valid[:, None]
    hist = jnp.sum(mask, axis=0)

    lane1 = jax.lax.iota(jnp.int32, 128)
    extra = jnp.where(lane1 == _BINS, jnp.sum(pv * valid),
                      jnp.where(lane1 == _BINS + 1, jnp.sum(valid), 0.0))
    acc_ref[0, :] += hist + extra


def kernel(decoder_out_att, ys_out_pad_att):
    b, t, v = decoder_out_att.shape
    n = b * t
    x = decoder_out_att.reshape(n, v)
    ys_flat = ys_out_pad_att.reshape(n)
    ys3 = ys_out_pad_att.reshape(n // _R, 1, _R)
    grid_spec = pltpu.PrefetchScalarGridSpec(
        num_scalar_prefetch=1,
        grid=(n // _R,),
        in_specs=[pl.BlockSpec((_R, v), lambda i, *_: (i, 0)),
                  pl.BlockSpec((1, 1, _R), lambda i, *_: (i, 0, 0))],
        out_specs=pl.BlockSpec((1, 128), lambda i, *_: (0, 0)),
    )
    acc = pl.pallas_call(
        _stat_block,
        grid_spec=grid_spec,
        out_shape=jax.ShapeDtypeStruct((1, 128), jnp.float32),
        compiler_params=pltpu.CompilerParams(dimension_semantics=("arbitrary",)),
    )(ys_flat, x, ys3)[0]
    mean = acc[_BINS] / jnp.maximum(acc[_BINS + 1], 1.0)
    return jnp.concatenate([mean[None], acc[:_BINS]])
